# baseline (speedup 1.0000x reference)
import jax
import jax.numpy as jnp
from jax import lax
from jax.experimental import pallas as pl
from jax.experimental.pallas import tpu as pltpu
from jax.experimental.pallas import tpu_sc as plsc

N_NODES = 10000
N_EDGES = 320000
NPAD = 10240
NC = 2
NS = 16
NW = NC * NS
E_PER_TILE = N_EDGES // NW
CHUNK = NPAD // NS
LANES = 16


XROWS = 624


def _extract_body(x_hbm, x0_hbm, xrows_v, xcol_v, sem):
    c = lax.axis_index("c")
    s = lax.axis_index("s")

    lane = lax.iota(jnp.int32, 16)
    col0 = jnp.zeros((LANES,), jnp.int32)
    rem = N_NODES - NS * XROWS

    @pl.when(c == jnp.int32(0))
    def _():
        xcp = pltpu.async_copy(
            x_hbm.at[pl.ds(s * XROWS, XROWS), pl.ds(jnp.int32(0), 8)],
            xrows_v.at[pl.ds(jnp.int32(0), XROWS), :], sem)

        @pl.when(s == jnp.int32(0))
        def _():
            pltpu.sync_copy(
                x_hbm.at[pl.ds(NS * XROWS, rem), pl.ds(jnp.int32(0), 8)],
                xrows_v.at[pl.ds(jnp.int32(XROWS), rem), :])

        xcp.wait()

        @plsc.parallel_loop(jnp.int32(0), jnp.int32(XROWS // LANES),
                            jnp.int32(1), unroll=13)
        def xbody(j):
            off = j * LANES
            xcol_v[pl.ds(off, LANES)] = plsc.load_gather(
                xrows_v, [lane + off, col0])

        pltpu.sync_copy(xcol_v.at[pl.ds(jnp.int32(0), XROWS)],
                        x0_hbm.at[pl.ds(s * XROWS, XROWS)])

        @pl.when(s == jnp.int32(0))
        def _():
            xcol_v[pl.ds(jnp.int32(XROWS), LANES)] = plsc.load_gather(
                xrows_v, [lane + XROWS, col0])
            pltpu.sync_copy(xcol_v.at[pl.ds(jnp.int32(XROWS), rem)],
                            x0_hbm.at[pl.ds(NS * XROWS, rem)])


def _diff_body(x0_hbm, src_hbm, dst_hbm, diff_hbm,
               x0_v, src_v, dst_v, diff_v, sem):
    c = lax.axis_index("c")
    s = lax.axis_index("s")
    wid = c * NS + s
    base = wid * E_PER_TILE

    cps = [
        pltpu.async_copy(x0_hbm, x0_v, sem),
        pltpu.async_copy(src_hbm.at[pl.ds(base, E_PER_TILE)], src_v, sem),
        pltpu.async_copy(dst_hbm.at[pl.ds(base, E_PER_TILE)], dst_v, sem),
    ]
    for cp in cps:
        cp.wait()

    @plsc.parallel_loop(jnp.int32(0), jnp.int32(E_PER_TILE // LANES),
                        jnp.int32(1), unroll=25)
    def ebody(j):
        off = j * LANES
        srcv = src_v[pl.ds(off, LANES)]
        dstv = dst_v[pl.ds(off, LANES)]
        xs = plsc.load_gather(x0_v, [srcv])
        xd = plsc.load_gather(x0_v, [dstv])
        diff_v[pl.ds(off, LANES)] = xs - xd

    pltpu.sync_copy(diff_v, diff_hbm.at[pl.ds(base, E_PER_TILE)])


def _scatter_body(diff_hbm, dst_hbm, w_hbm, out_hbm,
                  diff_v, dst_v, w_v, acc_v, red_v, shared, sem):
    c = lax.axis_index("c")
    s = lax.axis_index("s")
    wid = c * NS + s
    base = wid * E_PER_TILE

    cps = [
        pltpu.async_copy(diff_hbm.at[pl.ds(base, E_PER_TILE)], diff_v, sem),
        pltpu.async_copy(dst_hbm.at[pl.ds(base, E_PER_TILE)], dst_v, sem),
        pltpu.async_copy(w_hbm.at[pl.ds(base, E_PER_TILE)], w_v, sem),
    ]

    zeros16 = jnp.zeros((LANES,), jnp.float32)

    @plsc.parallel_loop(jnp.int32(0), jnp.int32(NPAD // LANES),
                        jnp.int32(1), unroll=8)
    def zbody(j):
        acc_v[pl.ds(j * LANES, LANES)] = zeros16

    for cp in cps:
        cp.wait()

    @plsc.parallel_loop(jnp.int32(0), jnp.int32(E_PER_TILE // LANES),
                        jnp.int32(1), unroll=25)
    def ebody(j):
        off = j * LANES
        dstv = dst_v[pl.ds(off, LANES)]
        contrib = diff_v[pl.ds(off, LANES)] * w_v[pl.ds(off, LANES)]
        plsc.addupdate_scatter(acc_v, [dstv], contrib)

    pltpu.sync_copy(acc_v, shared.at[s])
    plsc.subcore_barrier()

    nbase = s * CHUNK
    for r in range(NS):
        pltpu.sync_copy(shared.at[jnp.int32(r), pl.ds(nbase, CHUNK)],
                        red_v.at[jnp.int32(r)])

    @plsc.parallel_loop(jnp.int32(0), jnp.int32(CHUNK // LANES),
                        jnp.int32(1), unroll=4)
    def rbody(j):
        off = j * LANES
        a = red_v[jnp.int32(0), pl.ds(off, LANES)]
        for r in range(1, NS):
            a = a + red_v[jnp.int32(r), pl.ds(off, LANES)]
        acc_v[pl.ds(off, LANES)] = a
    pltpu.sync_copy(acc_v.at[pl.ds(jnp.int32(0), CHUNK)],
                    out_hbm.at[pl.ds(c * NPAD + nbase, CHUNK)])


@jax.jit
def _sc_call(x, src, dst, w):
    mesh = plsc.VectorSubcoreMesh(core_axis_name="c", subcore_axis_name="s")
    x0 = pl.kernel(
        _extract_body,
        out_type=jax.ShapeDtypeStruct((N_NODES,), jnp.float32),
        mesh=mesh,
        compiler_params=pltpu.CompilerParams(
            needs_layout_passes=False, use_tc_tiling_on_sc=False),
        scratch_types=[
            pltpu.VMEM((XROWS + LANES, 8), jnp.float32),
            pltpu.VMEM((XROWS + LANES,), jnp.float32),
            pltpu.SemaphoreType.DMA,
        ],
    )(x)
    diff = pl.kernel(
        _diff_body,
        out_type=jax.ShapeDtypeStruct((N_EDGES,), jnp.float32),
        mesh=mesh,
        compiler_params=pltpu.CompilerParams(
            needs_layout_passes=False, use_tc_tiling_on_sc=False),
        scratch_types=[
            pltpu.VMEM((N_NODES,), jnp.float32),
            pltpu.VMEM((E_PER_TILE,), jnp.int32),
            pltpu.VMEM((E_PER_TILE,), jnp.int32),
            pltpu.VMEM((E_PER_TILE,), jnp.float32),
            pltpu.SemaphoreType.DMA,
        ],
    )(x0, src, dst)
    return pl.kernel(
        _scatter_body,
        out_type=jax.ShapeDtypeStruct((NC * NPAD,), jnp.float32),
        mesh=mesh,
        compiler_params=pltpu.CompilerParams(
            needs_layout_passes=False, use_tc_tiling_on_sc=False),
        scratch_types=[
            pltpu.VMEM((E_PER_TILE,), jnp.float32),
            pltpu.VMEM((E_PER_TILE,), jnp.int32),
            pltpu.VMEM((E_PER_TILE,), jnp.float32),
            pltpu.VMEM((NPAD,), jnp.float32),
            pltpu.VMEM((NS, CHUNK), jnp.float32),
            pltpu.VMEM_SHARED((NS, NPAD), jnp.float32),
            pltpu.SemaphoreType.DMA,
        ],
    )(diff, dst, w)


def _combine_body(p_ref, o_ref):
    o_ref[...] = (p_ref[pl.ds(0, N_NODES)] +
                  p_ref[pl.ds(NPAD, N_NODES)])


@jax.jit
def _combine(partials):
    return pl.pallas_call(
        _combine_body,
        out_shape=jax.ShapeDtypeStruct((N_NODES,), jnp.float32),
    )(partials)


def kernel(x, edge_index, edge_attr):
    src = edge_index[0].astype(jnp.int32)
    dst = edge_index[1].astype(jnp.int32)
    w = edge_attr[:, 0] + edge_attr[:, 1]
    partials = _sc_call(x, src, dst, w)
    return _combine(partials)


# speedup vs baseline: 1.0416x; 1.0416x over previous
import jax
import jax.numpy as jnp
from jax import lax
from jax.experimental import pallas as pl
from jax.experimental.pallas import tpu as pltpu
from jax.experimental.pallas import tpu_sc as plsc

N_NODES = 10000
N_EDGES = 320000
NPAD = 10240
NC = 2
NS = 16
NW = NC * NS
E_PER_TILE = N_EDGES // NW
CHUNK = NPAD // NS
LANES = 16


XROWS = 624


def _diff_body(x_hbm, src_hbm, dst_hbm, diff_hbm,
               x0_v, src_v, dst_v, diff_v, xrows_v, xcol_v, xsh, sem):
    c = lax.axis_index("c")
    s = lax.axis_index("s")
    wid = c * NS + s
    base = wid * E_PER_TILE

    lane = lax.iota(jnp.int32, 16)
    col0 = jnp.zeros((LANES,), jnp.int32)
    rem = N_NODES - NS * XROWS
    xcp = pltpu.async_copy(
        x_hbm.at[pl.ds(s * XROWS, XROWS), pl.ds(jnp.int32(0), 8)],
        xrows_v.at[pl.ds(jnp.int32(0), XROWS), :], sem)
    cps = [
        pltpu.async_copy(src_hbm.at[pl.ds(base, E_PER_TILE)], src_v, sem),
        pltpu.async_copy(dst_hbm.at[pl.ds(base, E_PER_TILE)], dst_v, sem),
    ]

    @pl.when(s == jnp.int32(0))
    def _():
        pltpu.sync_copy(
            x_hbm.at[pl.ds(NS * XROWS, rem), pl.ds(jnp.int32(0), 8)],
            xrows_v.at[pl.ds(jnp.int32(XROWS), rem), :])

    xcp.wait()

    @plsc.parallel_loop(jnp.int32(0), jnp.int32(XROWS // LANES),
                        jnp.int32(1), unroll=13)
    def xbody(j):
        off = j * LANES
        xcol_v[pl.ds(off, LANES)] = plsc.load_gather(
            xrows_v, [lane + off, col0])

    pltpu.sync_copy(xcol_v.at[pl.ds(jnp.int32(0), XROWS)],
                    xsh.at[pl.ds(s * XROWS, XROWS)])

    @pl.when(s == jnp.int32(0))
    def _():
        xcol_v[pl.ds(jnp.int32(XROWS), LANES)] = plsc.load_gather(
            xrows_v, [lane + XROWS, col0])
        pltpu.sync_copy(xcol_v.at[pl.ds(jnp.int32(XROWS), rem)],
                        xsh.at[pl.ds(NS * XROWS, rem)])

    plsc.subcore_barrier()
    pltpu.sync_copy(xsh, x0_v)
    for cp in cps:
        cp.wait()

    @plsc.parallel_loop(jnp.int32(0), jnp.int32(E_PER_TILE // LANES),
                        jnp.int32(1), unroll=25)
    def ebody(j):
        off = j * LANES
        srcv = src_v[pl.ds(off, LANES)]
        dstv = dst_v[pl.ds(off, LANES)]
        xs = plsc.load_gather(x0_v, [srcv])
        xd = plsc.load_gather(x0_v, [dstv])
        diff_v[pl.ds(off, LANES)] = xs - xd

    pltpu.sync_copy(diff_v, diff_hbm.at[pl.ds(base, E_PER_TILE)])


def _scatter_body(diff_hbm, dst_hbm, w_hbm, out_hbm,
                  diff_v, dst_v, w_v, acc_v, red_v, shared, sem):
    c = lax.axis_index("c")
    s = lax.axis_index("s")
    wid = c * NS + s
    base = wid * E_PER_TILE

    cps = [
        pltpu.async_copy(diff_hbm.at[pl.ds(base, E_PER_TILE)], diff_v, sem),
        pltpu.async_copy(dst_hbm.at[pl.ds(base, E_PER_TILE)], dst_v, sem),
        pltpu.async_copy(w_hbm.at[pl.ds(base, E_PER_TILE)], w_v, sem),
    ]

    zeros16 = jnp.zeros((LANES,), jnp.float32)

    @plsc.parallel_loop(jnp.int32(0), jnp.int32(NPAD // LANES),
                        jnp.int32(1), unroll=8)
    def zbody(j):
        acc_v[pl.ds(j * LANES, LANES)] = zeros16

    for cp in cps:
        cp.wait()

    @plsc.parallel_loop(jnp.int32(0), jnp.int32(E_PER_TILE // LANES),
                        jnp.int32(1), unroll=25)
    def ebody(j):
        off = j * LANES
        dstv = dst_v[pl.ds(off, LANES)]
        contrib = diff_v[pl.ds(off, LANES)] * w_v[pl.ds(off, LANES)]
        plsc.addupdate_scatter(acc_v, [dstv], contrib)

    pltpu.sync_copy(acc_v, shared.at[s])
    plsc.subcore_barrier()

    nbase = s * CHUNK
    rcps = [pltpu.async_copy(shared.at[jnp.int32(r), pl.ds(nbase, CHUNK)],
                             red_v.at[jnp.int32(r)], sem)
            for r in range(NS)]
    for rcp in rcps:
        rcp.wait()

    @plsc.parallel_loop(jnp.int32(0), jnp.int32(CHUNK // LANES),
                        jnp.int32(1), unroll=4)
    def rbody(j):
        off = j * LANES
        a = red_v[jnp.int32(0), pl.ds(off, LANES)]
        for r in range(1, NS):
            a = a + red_v[jnp.int32(r), pl.ds(off, LANES)]
        acc_v[pl.ds(off, LANES)] = a
    pltpu.sync_copy(acc_v.at[pl.ds(jnp.int32(0), CHUNK)],
                    out_hbm.at[pl.ds(c * NPAD + nbase, CHUNK)])


@jax.jit
def _sc_call(x, src, dst, w):
    mesh = plsc.VectorSubcoreMesh(core_axis_name="c", subcore_axis_name="s")
    diff = pl.kernel(
        _diff_body,
        out_type=jax.ShapeDtypeStruct((N_EDGES,), jnp.float32),
        mesh=mesh,
        compiler_params=pltpu.CompilerParams(
            needs_layout_passes=False, use_tc_tiling_on_sc=False),
        scratch_types=[
            pltpu.VMEM((N_NODES,), jnp.float32),
            pltpu.VMEM((E_PER_TILE,), jnp.int32),
            pltpu.VMEM((E_PER_TILE,), jnp.int32),
            pltpu.VMEM((E_PER_TILE,), jnp.float32),
            pltpu.VMEM((XROWS + LANES, 8), jnp.float32),
            pltpu.VMEM((XROWS + LANES,), jnp.float32),
            pltpu.VMEM_SHARED((N_NODES,), jnp.float32),
            pltpu.SemaphoreType.DMA,
        ],
    )(x, src, dst)
    return pl.kernel(
        _scatter_body,
        out_type=jax.ShapeDtypeStruct((NC * NPAD,), jnp.float32),
        mesh=mesh,
        compiler_params=pltpu.CompilerParams(
            needs_layout_passes=False, use_tc_tiling_on_sc=False),
        scratch_types=[
            pltpu.VMEM((E_PER_TILE,), jnp.float32),
            pltpu.VMEM((E_PER_TILE,), jnp.int32),
            pltpu.VMEM((E_PER_TILE,), jnp.float32),
            pltpu.VMEM((NPAD,), jnp.float32),
            pltpu.VMEM((NS, CHUNK), jnp.float32),
            pltpu.VMEM_SHARED((NS, NPAD), jnp.float32),
            pltpu.SemaphoreType.DMA,
        ],
    )(diff, dst, w)


def _combine_body(p_ref, o_ref):
    o_ref[...] = (p_ref[pl.ds(0, N_NODES)] +
                  p_ref[pl.ds(NPAD, N_NODES)])


@jax.jit
def _combine(partials):
    return pl.pallas_call(
        _combine_body,
        out_shape=jax.ShapeDtypeStruct((N_NODES,), jnp.float32),
    )(partials)


def kernel(x, edge_index, edge_attr):
    src = edge_index[0].astype(jnp.int32)
    dst = edge_index[1].astype(jnp.int32)
    w = edge_attr[:, 0] + edge_attr[:, 1]
    partials = _sc_call(x, src, dst, w)
    return _combine(partials)
